# baseline (speedup 1.0000x reference)
import functools

import jax
import jax.numpy as jnp
from jax import lax
from jax.experimental import pallas as pl
from jax.experimental.pallas import tpu as pltpu
from jax.experimental.pallas import tpu_sc as plsc

HIDDEN = 64
B_TOTAL = 4096 * 200
NC, NS = 2, 16
NW = NC * NS
BPW = B_TOTAL // NW
G = 128
CH = 1024
GPC = CH // G
NCHUNK = BPW // CH


def _emb_body(x_hbm, table_hbm, out_hbm, idx_v, rows_v, gat_sem):
    wid = lax.axis_index("s") * NC + lax.axis_index("c")
    base = wid * BPW

    def chunk_body(c, carry):
        off = base + c * CH
        pltpu.sync_copy(x_hbm.at[wid, pl.ds(c * GPC, GPC)], idx_v)
        copies = []
        for j in range(GPC):
            copies.append(
                pltpu.async_copy(
                    table_hbm.at[idx_v.at[j]],
                    rows_v.at[pl.ds(j * G, G)],
                    gat_sem,
                )
            )
        for cp in copies:
            cp.wait()
        pltpu.sync_copy(rows_v, out_hbm.at[pl.ds(off, CH)])
        return carry

    lax.fori_loop(0, NCHUNK, chunk_body, 0)


@jax.jit
def _embed(x_flat2d, table):
    mesh = plsc.VectorSubcoreMesh(core_axis_name="c", subcore_axis_name="s")
    k = pl.kernel(
        _emb_body,
        out_type=jax.ShapeDtypeStruct((B_TOTAL, HIDDEN), jnp.float32),
        mesh=mesh,
        compiler_params=pltpu.CompilerParams(use_tc_tiling_on_sc=False),
        scratch_types=[
            pltpu.VMEM((GPC, G), jnp.int32),
            pltpu.VMEM((CH, HIDDEN), jnp.float32),
            pltpu.SemaphoreType.DMA,
        ],
    )
    return k(x_flat2d, table)


def kernel(x, table):
    b, h = x.shape
    x_flat = x.reshape(NW, BPW // G, G)
    out = _embed(x_flat, table)
    return out.reshape(b, h, HIDDEN)


# speedup vs baseline: 1.0164x; 1.0164x over previous
import functools

import jax
import jax.numpy as jnp
from jax import lax
from jax.experimental import pallas as pl
from jax.experimental.pallas import tpu as pltpu
from jax.experimental.pallas import tpu_sc as plsc

HIDDEN = 64
B_TOTAL = 4096 * 200
NC, NS = 2, 16
NW = NC * NS
BPW = B_TOTAL // NW
G = 128
CH = 512
GPC = CH // G
NCHUNK = BPW // CH
NPAIR = NCHUNK // 2


def _emb_body(x_hbm, table_hbm, out_hbm,
              idx_v, rows0, rows1, g0, g1, s0, s1):
    wid = lax.axis_index("s") * NC + lax.axis_index("c")
    base = wid * BPW
    rows = (rows0, rows1)
    gsem = (g0, g1)
    ssem = (s0, s1)

    def fire_gathers(pair, b):
        for j in range(GPC):
            pltpu.async_copy(
                table_hbm.at[idx_v.at[pair % 2, b * GPC + j]],
                rows[b].at[pl.ds(j * G, G)],
                gsem[b],
            )

    def wait_gathers(pair, b):
        for j in range(GPC):
            pltpu.make_async_copy(
                table_hbm.at[idx_v.at[pair % 2, b * GPC + j]],
                rows[b].at[pl.ds(j * G, G)],
                gsem[b],
            ).wait()

    def load_idx(pair):
        pltpu.sync_copy(x_hbm.at[wid, pl.ds(pair * 2 * GPC, 2 * GPC)],
                        idx_v.at[pair % 2])

    def store_descr(pair, b):
        off = base + (2 * pair + b) * CH
        return pltpu.make_async_copy(rows[b], out_hbm.at[pl.ds(off, CH)],
                                     ssem[b])

    load_idx(0)
    fire_gathers(0, 0)
    fire_gathers(0, 1)

    def pair_body(p, carry):
        @pl.when(p < NPAIR - 1)
        def _():
            load_idx(p + 1)

        for b in range(2):
            wait_gathers(p, b)
            store_descr(p, b).start()

        @pl.when(p < NPAIR - 1)
        def _():
            for b in range(2):
                store_descr(p, b).wait()
                fire_gathers(p + 1, b)
        return carry

    lax.fori_loop(0, NPAIR, pair_body, 0)

    for b in range(2):
        store_descr(NPAIR - 1, b).wait()


@jax.jit
def _embed(x3d, table):
    mesh = plsc.VectorSubcoreMesh(core_axis_name="c", subcore_axis_name="s")
    k = pl.kernel(
        _emb_body,
        out_type=jax.ShapeDtypeStruct((B_TOTAL, HIDDEN), jnp.float32),
        mesh=mesh,
        compiler_params=pltpu.CompilerParams(use_tc_tiling_on_sc=False),
        scratch_types=[
            pltpu.VMEM((2, 2 * GPC, G), jnp.int32),
            pltpu.VMEM((CH, HIDDEN), jnp.float32),
            pltpu.VMEM((CH, HIDDEN), jnp.float32),
            pltpu.SemaphoreType.DMA,
            pltpu.SemaphoreType.DMA,
            pltpu.SemaphoreType.DMA,
            pltpu.SemaphoreType.DMA,
        ],
    )
    return k(x3d, table)


def kernel(x, table):
    b, h = x.shape
    x3d = x.reshape(NW, BPW // G, G)
    out = _embed(x3d, table)
    return out.reshape(b, h, HIDDEN)
